# baseline (speedup 1.0000x reference)
import functools

import jax
import jax.numpy as jnp
from jax import lax
from jax.experimental import pallas as pl
from jax.experimental.pallas import tpu as pltpu
from jax.experimental.pallas import tpu_sc as plsc

B = 4096
L = 200
DIM = 64
NW = 32
CH = 128
TOTAL = B * L
B_PER_W = TOTAL // NW
N_CHUNKS = B_PER_W // CH


@functools.partial(jax.jit, static_argnames=())
def _gather(xf, table):
    mesh = plsc.VectorSubcoreMesh(core_axis_name="c", subcore_axis_name="s")
    nc = 2

    @functools.partial(
        pl.kernel,
        out_type=jax.ShapeDtypeStruct((NW, N_CHUNKS, CH, DIM), jnp.float32),
        mesh=mesh,
        scratch_types=[
            pltpu.VMEM((N_CHUNKS, CH), jnp.int32),
            pltpu.VMEM((CH, DIM), jnp.float32),
            pltpu.SemaphoreType.DMA,
        ],
        compiler_params=pltpu.CompilerParams(use_tc_tiling_on_sc=False),
    )
    def k(x_hbm, table_hbm, out_hbm, idx_v, rows_v, sem):
        wid = lax.axis_index("s") * nc + lax.axis_index("c")
        pltpu.sync_copy(x_hbm.at[wid], idx_v)

        def chunk(j, _):
            pltpu.async_copy(table_hbm.at[idx_v.at[j]], rows_v, sem).wait()
            pltpu.sync_copy(rows_v, out_hbm.at[wid, j])
            return 0

        lax.fori_loop(0, N_CHUNKS, chunk, 0, unroll=False)

    return k(xf, table)


def kernel(x, table):
    xf = x.reshape(NW, N_CHUNKS, CH)
    out = _gather(xf, table)
    return out.reshape(B, L, DIM)


# speedup vs baseline: 1.1139x; 1.1139x over previous
import functools

import jax
import jax.numpy as jnp
from jax import lax
from jax.experimental import pallas as pl
from jax.experimental.pallas import tpu as pltpu
from jax.experimental.pallas import tpu_sc as plsc

B = 4096
L = 200
DIM = 64
NW = 32
CH = 128
CPG = 4
G = CH * CPG
TOTAL = B * L
B_PER_W = TOTAL // NW
N_CHUNKS = B_PER_W // CH
NG = B_PER_W // G


def _gather(xf, table):
    mesh = plsc.VectorSubcoreMesh(core_axis_name="c", subcore_axis_name="s")
    nc = 2

    @functools.partial(
        pl.kernel,
        out_type=jax.ShapeDtypeStruct((NW, NG, G, DIM), jnp.float32),
        mesh=mesh,
        scratch_types=[
            pltpu.VMEM((N_CHUNKS, CH), jnp.int32),
            pltpu.VMEM((2, G, DIM), jnp.float32),
            pltpu.SemaphoreType.DMA,
            pltpu.SemaphoreType.DMA,
        ],
        compiler_params=pltpu.CompilerParams(use_tc_tiling_on_sc=False),
    )
    def k(x_hbm, table_hbm, out_hbm, idx_v, rows_v, gsem, wsem):
        wid = lax.axis_index("s") * nc + lax.axis_index("c")
        pltpu.sync_copy(x_hbm.at[wid], idx_v)

        def fire_gathers(g, p):
            for c in range(CPG):
                pltpu.async_copy(
                    table_hbm.at[idx_v.at[g * CPG + c]],
                    rows_v.at[p, pl.ds(c * CH, CH)],
                    gsem,
                )

        def drain_gathers(p):
            for c in range(CPG):
                pltpu.make_async_copy(
                    table_hbm.at[idx_v.at[c]],
                    rows_v.at[p, pl.ds(c * CH, CH)],
                    gsem,
                ).wait()

        def drain_wb(p):
            pltpu.make_async_copy(rows_v.at[p], out_hbm.at[wid, 0], wsem).wait()

        fire_gathers(0, 0)

        def outer(Gi, _):
            for p in range(2):
                g = Gi * 2 + p
                drain_gathers(p)
                pltpu.async_copy(rows_v.at[p], out_hbm.at[wid, g], wsem)
                @pl.when(g > 0)
                def _():
                    drain_wb(1 - p)
                @pl.when(g + 1 < NG)
                def _():
                    fire_gathers(g + 1, 1 - p)
            return 0

        lax.fori_loop(0, NG // 2, outer, 0, unroll=False)
        drain_wb(1)

    return k(xf, table)


def kernel(x, table):
    xf = x.reshape(NW, N_CHUNKS, CH)
    out = _gather(xf, table)
    return out.reshape(B, L, DIM)


# speedup vs baseline: 1.3539x; 1.2155x over previous
import functools

import jax
import jax.numpy as jnp
from jax import lax
from jax.experimental import pallas as pl
from jax.experimental.pallas import tpu as pltpu
from jax.experimental.pallas import tpu_sc as plsc

B = 4096
L = 200
DIM = 64
PD = 128
NW = 32
CH = 128
CPG = 2
G = CH * CPG
TOTAL = B * L
B_PER_W = TOTAL // NW
N_CHUNKS = B_PER_W // CH
NG = B_PER_W // G


def _gather(xf, table_p):
    mesh = plsc.VectorSubcoreMesh(core_axis_name="c", subcore_axis_name="s")
    nc = 2

    @functools.partial(
        pl.kernel,
        out_type=jax.ShapeDtypeStruct((NW, B_PER_W, PD), jnp.float32),
        mesh=mesh,
        scratch_types=[
            pltpu.VMEM((N_CHUNKS, CH), jnp.int32),
            pltpu.VMEM((2, G, PD), jnp.float32),
            pltpu.SemaphoreType.DMA,
            pltpu.SemaphoreType.DMA,
        ],
    )
    def k(x_hbm, table_hbm, out_hbm, idx_v, rows_v, gsem, wsem):
        wid = lax.axis_index("s") * nc + lax.axis_index("c")
        pltpu.sync_copy(x_hbm.at[wid], idx_v)

        def fire_gathers(g, p):
            for c in range(CPG):
                pltpu.async_copy(
                    table_hbm.at[idx_v.at[g * CPG + c]],
                    rows_v.at[p, pl.ds(c * CH, CH)],
                    gsem,
                )

        def drain_gathers(p):
            for c in range(CPG):
                pltpu.make_async_copy(
                    table_hbm.at[idx_v.at[c]],
                    rows_v.at[p, pl.ds(c * CH, CH)],
                    gsem,
                ).wait()

        def fire_wb(g, p):
            pltpu.async_copy(
                rows_v.at[p],
                out_hbm.at[wid, pl.ds(g * G, G)],
                wsem,
            )

        def drain_wb(g, p):
            pltpu.make_async_copy(
                rows_v.at[p],
                out_hbm.at[wid, pl.ds(g * G, G)],
                wsem,
            ).wait()

        fire_gathers(0, 0)

        def outer(Gi, _):
            for p in range(2):
                g = Gi * 2 + p
                drain_gathers(p)
                fire_wb(g, p)
                @pl.when(g > 0)
                def _():
                    drain_wb(g - 1, 1 - p)
                @pl.when(g + 1 < NG)
                def _():
                    fire_gathers(g + 1, 1 - p)
            return 0

        lax.fori_loop(0, NG // 2, outer, 0, unroll=False)
        drain_wb(NG - 1, 1)

    return k(xf, table_p)


def kernel(x, table):
    table_p = jnp.pad(table, ((0, 0), (0, PD - DIM)))
    xf = x.reshape(NW, N_CHUNKS, CH)
    out = _gather(xf, table_p)
    return out[:, :, :DIM].reshape(B, L, DIM)
